# baseline (speedup 1.0000x reference)
import functools

import jax
import jax.numpy as jnp
from jax import lax
from jax.experimental import pallas as pl
from jax.experimental.pallas import tpu as pltpu
from jax.experimental.pallas import tpu_sc as plsc

N = 10000
E = 320000
D_IN = 128
D_OUT = 9
L = 16

NC = 2
NS = 16
NW = NC * NS
CH = 128
NCHUNK = 80
E_PAD = NW * NCHUNK * CH
N_PAD = 10112
RPT = N_PAD // NS


def _proj_body(x_ref, wl_ref, wr_ref, bl_ref, y_ref, z_ref):
    x = x_ref[...]
    y = jnp.dot(x, wl_ref[...], preferred_element_type=jnp.float32)
    row = lax.broadcasted_iota(jnp.int32, (N_PAD, L), 0)
    col = lax.broadcasted_iota(jnp.int32, (N_PAD, L), 1)
    y_ref[...] = jnp.where((col == L - 1) & (row < N), 1.0, y)
    z_ref[...] = jnp.dot(x, wr_ref[...], preferred_element_type=jnp.float32) + bl_ref[...]


NBUF = 4


def _sc_body(y_hbm, src_hbm, dst_hbm, zero_hbm, out_hbm,
             src_v, dst_v, bufs, y_s, acc_s, gsems, ssems):
    cid = lax.axis_index("c")
    sid = lax.axis_index("s")
    tid = cid * NS + sid

    row_sl = pl.ds(sid * RPT, RPT)
    pltpu.async_copy(zero_hbm.at[row_sl], acc_s.at[row_sl], gsems[0])
    pltpu.async_copy(y_hbm.at[row_sl], y_s.at[row_sl], gsems[1])
    pltpu.async_copy(src_hbm.at[tid], src_v, gsems[2])
    pltpu.async_copy(dst_hbm.at[tid], dst_v, gsems[3])
    pltpu.make_async_copy(zero_hbm.at[row_sl], acc_s.at[row_sl], gsems[0]).wait()
    pltpu.make_async_copy(y_hbm.at[row_sl], y_s.at[row_sl], gsems[1]).wait()
    pltpu.make_async_copy(src_hbm.at[tid], src_v, gsems[2]).wait()
    pltpu.make_async_copy(dst_hbm.at[tid], dst_v, gsems[3]).wait()
    plsc.subcore_barrier()

    def src_table(b):
        return y_hbm if b == NBUF - 1 else y_s

    def gather(j, b):
        pltpu.async_copy(src_table(b).at[src_v.at[j]], bufs[b], gsems[b])

    def wait_gather(j, b):
        pltpu.make_async_copy(src_table(b).at[src_v.at[j]], bufs[b],
                              gsems[b]).wait()

    def scatter(j, b):
        return pltpu.async_copy(bufs[b], acc_s.at[dst_v.at[j]], ssems[b],
                                add=True)

    for b in range(NBUF):
        gather(b, b)

    def body(i, carry):
        descs = []
        for b in range(NBUF):
            j = NBUF * i + b
            wait_gather(j, b)
            descs.append(scatter(j, b))
        for b in range(NBUF):
            descs[b].wait()
            gather(NBUF * i + NBUF + b, b)
        return carry

    lax.fori_loop(0, NCHUNK // NBUF - 1, body, 0)
    tail = []
    for b in range(NBUF):
        j = NCHUNK - NBUF + b
        wait_gather(j, b)
        tail.append(scatter(j, b))
    for d in tail:
        d.wait()
    plsc.subcore_barrier()
    pltpu.sync_copy(acc_s.at[pl.ds(sid * RPT, RPT)],
                    out_hbm.at[cid, pl.ds(sid * RPT, RPT)])


def _fin_body(p_ref, z_ref, o_ref):
    p = p_ref[0] + p_ref[1]
    cnt = p[:, L - 1:L]
    u = p / jnp.maximum(cnt, 1.0) + z_ref[...]
    col = lax.broadcasted_iota(jnp.int32, (N_PAD, L), 1)
    u = jnp.where(col < D_OUT, u, -jnp.inf)
    m = jnp.max(u, axis=1, keepdims=True)
    lse = jnp.log(jnp.sum(jnp.exp(u - m), axis=1, keepdims=True)) + m
    o_ref[...] = u - lse


def kernel(x, edge_index, W_l, b_l, W_r):
    f32 = jnp.float32
    src = edge_index[0].astype(jnp.int32)
    dst = edge_index[1].astype(jnp.int32)
    pad = jnp.full((E_PAD - E,), N, jnp.int32)
    src_g = jnp.concatenate([src, pad]).reshape(NW, NCHUNK, CH)
    dst_g = jnp.concatenate([dst, pad]).reshape(NW, NCHUNK, CH)

    x_pad = jnp.zeros((N_PAD, D_IN), f32).at[:N].set(x)
    wl_pad = jnp.zeros((D_IN, L), f32).at[:, :D_OUT].set(W_l)
    wr_pad = jnp.zeros((D_IN, L), f32).at[:, :D_OUT].set(W_r)
    bl_pad = jnp.zeros((1, L), f32).at[0, :D_OUT].set(b_l)

    y_pad, z_pad = pl.pallas_call(
        _proj_body,
        out_shape=[jax.ShapeDtypeStruct((N_PAD, L), f32),
                   jax.ShapeDtypeStruct((N_PAD, L), f32)],
    )(x_pad, wl_pad, wr_pad, bl_pad)

    sc_call = pl.kernel(
        _sc_body,
        out_type=jax.ShapeDtypeStruct((NC, N_PAD, L), f32),
        mesh=plsc.VectorSubcoreMesh(core_axis_name="c", subcore_axis_name="s"),
        compiler_params=pltpu.CompilerParams(use_tc_tiling_on_sc=False),
        scratch_types=[
            pltpu.VMEM((NCHUNK, CH), jnp.int32),
            pltpu.VMEM((NCHUNK, CH), jnp.int32),
            [pltpu.VMEM((CH, L), f32)] * NBUF,
            pltpu.VMEM_SHARED((N_PAD, L), f32),
            pltpu.VMEM_SHARED((N_PAD, L), f32),
            [pltpu.SemaphoreType.DMA] * NBUF,
            [pltpu.SemaphoreType.DMA] * NBUF,
        ],
    )
    partials = sc_call(y_pad, src_g, dst_g, jnp.zeros((N_PAD, L), f32))

    out_pad = pl.pallas_call(
        _fin_body,
        out_shape=jax.ShapeDtypeStruct((N_PAD, L), f32),
    )(partials, z_pad)
    return out_pad[:N, :D_OUT]


# speedup vs baseline: 1.0345x; 1.0345x over previous
import functools

import jax
import jax.numpy as jnp
from jax import lax
from jax.experimental import pallas as pl
from jax.experimental.pallas import tpu as pltpu
from jax.experimental.pallas import tpu_sc as plsc

N = 10000
E = 320000
D_IN = 128
D_OUT = 9
L = 16

NC = 2
NS = 16
NW = NC * NS
CH = 128
NCHUNK = 80
E_PAD = NW * NCHUNK * CH
N_PAD = 10112
RPT = N_PAD // NS


def _proj_body(x_ref, wl_ref, wr_ref, bl_ref, y_ref, z_ref):
    x = x_ref[...]
    y = jnp.dot(x, wl_ref[...], preferred_element_type=jnp.float32)
    row = lax.broadcasted_iota(jnp.int32, (N_PAD, L), 0)
    col = lax.broadcasted_iota(jnp.int32, (N_PAD, L), 1)
    y_ref[...] = jnp.where((col == L - 1) & (row < N), 1.0, y)
    z_ref[...] = jnp.dot(x, wr_ref[...], preferred_element_type=jnp.float32) + bl_ref[...]


NBUF = 4


def _sc_body(y_hbm, src_hbm, dst_hbm, zero_hbm, out_hbm,
             src_v, dst_v, bufs, y_s, acc_s, gsems, ssems):
    cid = lax.axis_index("c")
    sid = lax.axis_index("s")
    tid = cid * NS + sid

    row_sl = pl.ds(sid * RPT, RPT)
    pltpu.async_copy(zero_hbm.at[row_sl], acc_s.at[row_sl], gsems[0])
    pltpu.async_copy(y_hbm.at[row_sl], y_s.at[row_sl], gsems[1])
    pltpu.async_copy(src_hbm.at[tid], src_v, gsems[2])
    pltpu.async_copy(dst_hbm.at[tid], dst_v, gsems[3])
    pltpu.make_async_copy(zero_hbm.at[row_sl], acc_s.at[row_sl], gsems[0]).wait()
    pltpu.make_async_copy(y_hbm.at[row_sl], y_s.at[row_sl], gsems[1]).wait()
    pltpu.make_async_copy(src_hbm.at[tid], src_v, gsems[2]).wait()
    pltpu.make_async_copy(dst_hbm.at[tid], dst_v, gsems[3]).wait()
    plsc.subcore_barrier()

    def gather(j, b):
        pltpu.async_copy(y_s.at[src_v.at[j]], bufs[b], gsems[b])

    def wait_gather(j, b):
        pltpu.make_async_copy(y_s.at[src_v.at[j]], bufs[b], gsems[b]).wait()

    def scatter(j, b):
        return pltpu.async_copy(bufs[b], acc_s.at[dst_v.at[j]], ssems[b],
                                add=True)

    for b in range(NBUF):
        gather(b, b)

    def body(i, carry):
        descs = []
        for b in range(NBUF):
            j = NBUF * i + b
            wait_gather(j, b)
            descs.append(scatter(j, b))
        for b in range(NBUF):
            descs[b].wait()
            gather(NBUF * i + NBUF + b, b)
        return carry

    lax.fori_loop(0, NCHUNK // NBUF - 1, body, 0)
    tail = []
    for b in range(NBUF):
        j = NCHUNK - NBUF + b
        wait_gather(j, b)
        tail.append(scatter(j, b))
    for d in tail:
        d.wait()
    plsc.subcore_barrier()
    pltpu.sync_copy(acc_s.at[pl.ds(sid * RPT, RPT)],
                    out_hbm.at[cid, pl.ds(sid * RPT, RPT)])


def _fin_body(p_ref, z_ref, o_ref):
    p = p_ref[0] + p_ref[1]
    cnt = p[:, L - 1:L]
    u = p / jnp.maximum(cnt, 1.0) + z_ref[...]
    col = lax.broadcasted_iota(jnp.int32, (N_PAD, L), 1)
    u = jnp.where(col < D_OUT, u, -jnp.inf)
    m = jnp.max(u, axis=1, keepdims=True)
    lse = jnp.log(jnp.sum(jnp.exp(u - m), axis=1, keepdims=True)) + m
    o_ref[...] = u - lse


def kernel(x, edge_index, W_l, b_l, W_r):
    f32 = jnp.float32
    src = edge_index[0].astype(jnp.int32)
    dst = edge_index[1].astype(jnp.int32)
    pad = jnp.full((E_PAD - E,), N, jnp.int32)
    src_g = jnp.concatenate([src, pad]).reshape(NW, NCHUNK, CH)
    dst_g = jnp.concatenate([dst, pad]).reshape(NW, NCHUNK, CH)

    x_pad = jnp.zeros((N_PAD, D_IN), f32).at[:N].set(x)
    wl_pad = jnp.zeros((D_IN, L), f32).at[:, :D_OUT].set(W_l)
    wr_pad = jnp.zeros((D_IN, L), f32).at[:, :D_OUT].set(W_r)
    bl_pad = jnp.zeros((1, L), f32).at[0, :D_OUT].set(b_l)

    y_pad, z_pad = pl.pallas_call(
        _proj_body,
        out_shape=[jax.ShapeDtypeStruct((N_PAD, L), f32),
                   jax.ShapeDtypeStruct((N_PAD, L), f32)],
    )(x_pad, wl_pad, wr_pad, bl_pad)

    sc_call = pl.kernel(
        _sc_body,
        out_type=jax.ShapeDtypeStruct((NC, N_PAD, L), f32),
        mesh=plsc.VectorSubcoreMesh(core_axis_name="c", subcore_axis_name="s"),
        compiler_params=pltpu.CompilerParams(use_tc_tiling_on_sc=False),
        scratch_types=[
            pltpu.VMEM((NCHUNK, CH), jnp.int32),
            pltpu.VMEM((NCHUNK, CH), jnp.int32),
            [pltpu.VMEM((CH, L), f32)] * NBUF,
            pltpu.VMEM_SHARED((N_PAD, L), f32),
            pltpu.VMEM_SHARED((N_PAD, L), f32),
            [pltpu.SemaphoreType.DMA] * NBUF,
            [pltpu.SemaphoreType.DMA] * NBUF,
        ],
    )
    partials = sc_call(y_pad, src_g, dst_g, jnp.zeros((N_PAD, L), f32))

    out_pad = pl.pallas_call(
        _fin_body,
        out_shape=jax.ShapeDtypeStruct((N_PAD, L), f32),
    )(partials, z_pad)
    return out_pad[:N, :D_OUT]


# speedup vs baseline: 1.0706x; 1.0350x over previous
import functools

import jax
import jax.numpy as jnp
from jax import lax
from jax.experimental import pallas as pl
from jax.experimental.pallas import tpu as pltpu
from jax.experimental.pallas import tpu_sc as plsc

N = 10000
E = 320000
D_IN = 128
D_OUT = 9
L = 16

NC = 2
NS = 16
NW = NC * NS
CH = 125
NCHUNK = 80
N_PAD = 10112
RPT = N_PAD // NS


def _proj_body(x_ref, wl_ref, wr_ref, bl_ref, y_ref, z_ref):
    x = x_ref[...]
    y = jnp.dot(x, wl_ref[...], preferred_element_type=jnp.float32)
    col = lax.broadcasted_iota(jnp.int32, (N, L), 1)
    y_ref[0:N] = jnp.where(col == L - 1, 1.0, y)
    y_ref[N:N_PAD] = jnp.zeros((N_PAD - N, L), jnp.float32)
    z_ref[...] = jnp.dot(x, wr_ref[...], preferred_element_type=jnp.float32) + bl_ref[...]


NBUF = 4


def _sc_body(y_hbm, src_hbm, dst_hbm, zero_hbm, out_hbm,
             src_v, dst_v, bufs, y_s, acc_s, gsems, ssems):
    cid = lax.axis_index("c")
    sid = lax.axis_index("s")
    tid = cid * NS + sid

    row_sl = pl.ds(sid * RPT, RPT)
    pltpu.async_copy(zero_hbm.at[row_sl], acc_s.at[row_sl], gsems[0])
    pltpu.async_copy(y_hbm.at[row_sl], y_s.at[row_sl], gsems[1])
    pltpu.async_copy(src_hbm.at[tid], src_v, gsems[2])
    pltpu.async_copy(dst_hbm.at[tid], dst_v, gsems[3])
    pltpu.make_async_copy(zero_hbm.at[row_sl], acc_s.at[row_sl], gsems[0]).wait()
    pltpu.make_async_copy(y_hbm.at[row_sl], y_s.at[row_sl], gsems[1]).wait()
    pltpu.make_async_copy(src_hbm.at[tid], src_v, gsems[2]).wait()
    pltpu.make_async_copy(dst_hbm.at[tid], dst_v, gsems[3]).wait()
    plsc.subcore_barrier()

    def gather(j, b):
        pltpu.async_copy(y_s.at[src_v.at[j]], bufs[b], gsems[b])

    def wait_gather(j, b):
        pltpu.make_async_copy(y_s.at[src_v.at[j]], bufs[b], gsems[b]).wait()

    def scatter(j, b):
        return pltpu.async_copy(bufs[b], acc_s.at[dst_v.at[j]], ssems[b],
                                add=True)

    for b in range(NBUF):
        gather(b, b)

    def body(i, carry):
        descs = []
        for b in range(NBUF):
            j = NBUF * i + b
            wait_gather(j, b)
            descs.append(scatter(j, b))
        for b in range(NBUF):
            descs[b].wait()
            gather(NBUF * i + NBUF + b, b)
        return carry

    lax.fori_loop(0, NCHUNK // NBUF - 1, body, 0)
    tail = []
    for b in range(NBUF):
        j = NCHUNK - NBUF + b
        wait_gather(j, b)
        tail.append(scatter(j, b))
    for d in tail:
        d.wait()
    plsc.subcore_barrier()
    pltpu.sync_copy(acc_s.at[pl.ds(sid * RPT, RPT)],
                    out_hbm.at[cid, pl.ds(sid * RPT, RPT)])


def _fin_body(p_ref, z_ref, o_ref):
    p = p_ref[0, 0:N] + p_ref[1, 0:N]
    cnt = p[:, L - 1:L]
    u = p / jnp.maximum(cnt, 1.0) + z_ref[...]
    col = lax.broadcasted_iota(jnp.int32, (N, L), 1)
    u = jnp.where(col < D_OUT, u, -jnp.inf)
    m = jnp.max(u, axis=1, keepdims=True)
    lse = jnp.log(jnp.sum(jnp.exp(u - m), axis=1, keepdims=True)) + m
    o_ref[...] = (u - lse)[:, 0:D_OUT]


def kernel(x, edge_index, W_l, b_l, W_r):
    f32 = jnp.float32
    src_g = edge_index[0].astype(jnp.int32).reshape(NW, NCHUNK, CH)
    dst_g = edge_index[1].astype(jnp.int32).reshape(NW, NCHUNK, CH)

    wl_pad = jnp.zeros((D_IN, L), f32).at[:, :D_OUT].set(W_l)
    wr_pad = jnp.zeros((D_IN, L), f32).at[:, :D_OUT].set(W_r)
    bl_pad = jnp.zeros((1, L), f32).at[0, :D_OUT].set(b_l)

    y_pad, z_out = pl.pallas_call(
        _proj_body,
        out_shape=[jax.ShapeDtypeStruct((N_PAD, L), f32),
                   jax.ShapeDtypeStruct((N, L), f32)],
    )(x, wl_pad, wr_pad, bl_pad)

    sc_call = pl.kernel(
        _sc_body,
        out_type=jax.ShapeDtypeStruct((NC, N_PAD, L), f32),
        mesh=plsc.VectorSubcoreMesh(core_axis_name="c", subcore_axis_name="s"),
        compiler_params=pltpu.CompilerParams(use_tc_tiling_on_sc=False),
        scratch_types=[
            pltpu.VMEM((NCHUNK, CH), jnp.int32),
            pltpu.VMEM((NCHUNK, CH), jnp.int32),
            [pltpu.VMEM((CH, L), f32)] * NBUF,
            pltpu.VMEM_SHARED((N_PAD, L), f32),
            pltpu.VMEM_SHARED((N_PAD, L), f32),
            [pltpu.SemaphoreType.DMA] * NBUF,
            [pltpu.SemaphoreType.DMA] * NBUF,
        ],
    )
    partials = sc_call(y_pad, src_g, dst_g, jnp.zeros((N_PAD, L), f32))

    return pl.pallas_call(
        _fin_body,
        out_shape=jax.ShapeDtypeStruct((N, D_OUT), f32),
    )(partials, z_out)


# speedup vs baseline: 1.2110x; 1.1311x over previous
import functools

import jax
import jax.numpy as jnp
from jax import lax
from jax.experimental import pallas as pl
from jax.experimental.pallas import tpu as pltpu
from jax.experimental.pallas import tpu_sc as plsc

N = 10000
E = 320000
D_IN = 128
D_OUT = 9
L = 16

NC = 2
NS = 16
NW = NC * NS
CH = 125
NCHUNK = 80
N_PAD = 10112
RPT = N_PAD // NS
NPK = N_PAD * L // 128


PKB = 128 // L


def _proj_body(x_ref, wl_ref, wr_ref, bl_ref, si_ref, di_ref,
               y_ref, z_ref, so_ref, do_ref):
    f32 = jnp.float32
    x = x_ref[...]
    wl = wl_ref[...]
    wr = wr_ref[...]
    bl = bl_ref[...]
    col = lax.broadcasted_iota(jnp.int32, (NPK, L), 1)
    tail = jnp.zeros((NPK * PKB - N, D_IN), f32)
    for k in range(PKB):
        if k < PKB - 1:
            xk = x[NPK * k:NPK * (k + 1)]
        else:
            xk = jnp.concatenate([x[NPK * k:N], tail], 0)
        yk = jnp.dot(xk, wl, preferred_element_type=f32)
        y_ref[:, L * k:L * (k + 1)] = jnp.where(col == L - 1, 1.0, yk)
        z_ref[:, L * k:L * (k + 1)] = (
            jnp.dot(xk, wr, preferred_element_type=f32) + bl)
    s = si_ref[...]
    ks = s // NPK
    so_ref[...] = (s - ks * NPK) * PKB + ks
    d = di_ref[...]
    kd = d // NPK
    do_ref[...] = (d - kd * NPK) * PKB + kd


NBUF = 4


def _sc_body(y_hbm, src_hbm, dst_hbm, zero_hbm, out_hbm,
             src_v, dst_v, bufs, y_s, acc_s, gsems, ssems):
    cid = lax.axis_index("c")
    sid = lax.axis_index("s")
    tid = cid * NS + sid

    row_sl = pl.ds(sid * RPT, RPT)
    pltpu.async_copy(zero_hbm.at[row_sl], acc_s.at[row_sl], gsems[0])
    pltpu.async_copy(y_hbm.at[row_sl], y_s.at[row_sl], gsems[1])
    pltpu.async_copy(src_hbm.at[tid], src_v, gsems[2])
    pltpu.async_copy(dst_hbm.at[tid], dst_v, gsems[3])
    pltpu.make_async_copy(zero_hbm.at[row_sl], acc_s.at[row_sl], gsems[0]).wait()
    pltpu.make_async_copy(y_hbm.at[row_sl], y_s.at[row_sl], gsems[1]).wait()
    pltpu.make_async_copy(src_hbm.at[tid], src_v, gsems[2]).wait()
    pltpu.make_async_copy(dst_hbm.at[tid], dst_v, gsems[3]).wait()
    plsc.subcore_barrier()

    def gather(j, b):
        pltpu.async_copy(y_s.at[src_v.at[j]], bufs[b], gsems[b])

    def wait_gather(j, b):
        pltpu.make_async_copy(y_s.at[src_v.at[j]], bufs[b], gsems[b]).wait()

    def scatter(j, b):
        return pltpu.async_copy(bufs[b], acc_s.at[dst_v.at[j]], ssems[b],
                                add=True)

    for b in range(NBUF):
        gather(b, b)

    def body(i, carry):
        descs = []
        for b in range(NBUF):
            j = NBUF * i + b
            wait_gather(j, b)
            descs.append(scatter(j, b))
        for b in range(NBUF):
            descs[b].wait()
            gather(NBUF * i + NBUF + b, b)
        return carry

    lax.fori_loop(0, NCHUNK // NBUF - 1, body, 0)
    tail = []
    for b in range(NBUF):
        j = NCHUNK - NBUF + b
        wait_gather(j, b)
        tail.append(scatter(j, b))
    for d in tail:
        d.wait()
    plsc.subcore_barrier()
    pltpu.sync_copy(acc_s.at[pl.ds(sid * RPT, RPT)],
                    out_hbm.at[cid, pl.ds(sid * RPT, RPT)])


def _fin_body(p_ref, z_ref, o_ref):
    f32 = jnp.float32
    p = p_ref[0] + p_ref[1]
    i2 = lax.broadcasted_iota(jnp.int32, (128, 128), 0)
    j2 = lax.broadcasted_iota(jnp.int32, (128, 128), 1)
    jg = (j2 // L) * L
    b_cnt = (i2 == jg + (L - 1)).astype(f32)
    b_head = (i2 == jg).astype(f32)
    b_sum = (i2 // L == j2 // L).astype(f32)
    cnt = jnp.dot(p, b_cnt, preferred_element_type=f32)
    u = p / jnp.maximum(cnt, 1.0) + z_ref[...]
    col = lax.broadcasted_iota(jnp.int32, (NPK, 128), 1)
    u = jnp.where(col % L < D_OUT, u, -jnp.inf)
    v = u
    for s in (1, 2, 4, 8):
        v = jnp.maximum(v, pltpu.roll(v, 128 - s, 1))
    m = jnp.dot(v, b_head, preferred_element_type=f32)
    e = jnp.exp(u - m)
    lse = jnp.log(jnp.dot(e, b_sum, preferred_element_type=f32)) + m
    o = u - lse
    for k in range(PKB - 1):
        o_ref[NPK * k:NPK * (k + 1), :] = o[:, L * k:L * k + D_OUT]
    o_ref[NPK * (PKB - 1):N, :] = (
        o[0:N - NPK * (PKB - 1), L * (PKB - 1):L * (PKB - 1) + D_OUT])


def kernel(x, edge_index, W_l, b_l, W_r):
    f32 = jnp.float32
    i32 = jnp.int32
    src_f = edge_index[0].astype(i32).reshape(NW * NCHUNK, CH)
    dst_f = edge_index[1].astype(i32).reshape(NW * NCHUNK, CH)

    wl_pad = jnp.zeros((D_IN, L), f32).at[:, :D_OUT].set(W_l)
    wr_pad = jnp.zeros((D_IN, L), f32).at[:, :D_OUT].set(W_r)
    bl_pad = jnp.zeros((1, L), f32).at[0, :D_OUT].set(b_l)

    y_pk, z_pk, src_t, dst_t = pl.pallas_call(
        _proj_body,
        out_shape=[jax.ShapeDtypeStruct((NPK, 128), f32),
                   jax.ShapeDtypeStruct((NPK, 128), f32),
                   jax.ShapeDtypeStruct((NW * NCHUNK, CH), i32),
                   jax.ShapeDtypeStruct((NW * NCHUNK, CH), i32)],
    )(x, wl_pad, wr_pad, bl_pad, src_f, dst_f)
    src_g = src_t.reshape(NW, NCHUNK, CH)
    dst_g = dst_t.reshape(NW, NCHUNK, CH)

    sc_call = pl.kernel(
        _sc_body,
        out_type=jax.ShapeDtypeStruct((NC, N_PAD, L), f32),
        mesh=plsc.VectorSubcoreMesh(core_axis_name="c", subcore_axis_name="s"),
        compiler_params=pltpu.CompilerParams(use_tc_tiling_on_sc=False),
        scratch_types=[
            pltpu.VMEM((NCHUNK, CH), jnp.int32),
            pltpu.VMEM((NCHUNK, CH), jnp.int32),
            [pltpu.VMEM((CH, L), f32)] * NBUF,
            pltpu.VMEM_SHARED((N_PAD, L), f32),
            pltpu.VMEM_SHARED((N_PAD, L), f32),
            [pltpu.SemaphoreType.DMA] * NBUF,
            [pltpu.SemaphoreType.DMA] * NBUF,
        ],
    )
    partials = sc_call(y_pk.reshape(N_PAD, L), src_g, dst_g,
                       jnp.zeros((N_PAD, L), f32))

    return pl.pallas_call(
        _fin_body,
        out_shape=jax.ShapeDtypeStruct((N, D_OUT), f32),
    )(partials.reshape(NC, NPK, 128), z_pk)
